# baseline (device time: 12166 ns/iter reference)
import jax
import jax.numpy as jnp
from jax import lax
from jax.experimental import pallas as pl
from jax.experimental.pallas import tpu as pltpu

N_DEV = 4
EPS = 1e-5


def kernel(x, Wp):
    b, h_loc, w, c = x.shape
    c_out = Wp.shape[1]
    n_global = N_DEV * h_loc * w

    def body(x_ref, wp_ref, out_ref, stats_ref, send_sems, recv_sems):
        my_pos = lax.axis_index("i")

        xf = x_ref[...].astype(jnp.float32)
        x2d = xf.reshape(b, h_loc * w, c)
        s1 = jnp.sum(x2d, axis=1)
        s2 = jnp.sum(x2d * x2d, axis=1)
        stats_ref[0, :, :] = jnp.concatenate([s1, s2], axis=0)

        barrier_sem = pltpu.get_barrier_semaphore()
        for k in range(1, N_DEV):
            peer = (my_pos + k) % N_DEV
            pl.semaphore_signal(
                barrier_sem, inc=1,
                device_id=(peer,), device_id_type=pl.DeviceIdType.MESH,
            )
        pl.semaphore_wait(barrier_sem, N_DEV - 1)

        rdmas = []
        for k in range(1, N_DEV):
            peer = (my_pos + k) % N_DEV
            rdma = pltpu.make_async_remote_copy(
                src_ref=stats_ref.at[0],
                dst_ref=stats_ref.at[k],
                send_sem=send_sems.at[k - 1],
                recv_sem=recv_sems.at[k - 1],
                device_id=(peer,),
                device_id_type=pl.DeviceIdType.MESH,
            )
            rdma.start()
            rdmas.append(rdma)
        for rdma in rdmas:
            rdma.wait_send()
        for rdma in rdmas:
            rdma.wait_recv()

        tot = (
            stats_ref[0, :, :] + stats_ref[1, :, :]
            + stats_ref[2, :, :] + stats_ref[3, :, :]
        )
        mean = tot[:b, :] / n_global
        var = tot[b:, :] / n_global - mean * mean
        inv = lax.rsqrt(var + EPS)
        h = (xf - mean.reshape(b, 1, 1, c)) * inv.reshape(b, 1, 1, c)
        a = h * jax.nn.sigmoid(h)
        a2 = a.reshape(b * h_loc * w, c).astype(jnp.bfloat16)
        wp = wp_ref[...].astype(jnp.bfloat16)
        o = jnp.dot(a2, wp, preferred_element_type=jnp.float32)
        out_ref[...] = o.reshape(b, h_loc, w, c_out)

    return pl.pallas_call(
        body,
        out_shape=jax.ShapeDtypeStruct((b, h_loc, w, c_out), jnp.float32),
        in_specs=[
            pl.BlockSpec(memory_space=pltpu.VMEM),
            pl.BlockSpec(memory_space=pltpu.VMEM),
        ],
        out_specs=pl.BlockSpec(memory_space=pltpu.VMEM),
        scratch_shapes=[
            pltpu.VMEM((N_DEV, 2 * b, c), jnp.float32),
            pltpu.SemaphoreType.DMA((N_DEV - 1,)),
            pltpu.SemaphoreType.DMA((N_DEV - 1,)),
        ],
        compiler_params=pltpu.CompilerParams(collective_id=0),
    )(x, Wp)


# device time: 11705 ns/iter; 1.0394x vs baseline; 1.0394x over previous
import jax
import jax.numpy as jnp
from jax import lax
from jax.experimental import pallas as pl
from jax.experimental.pallas import tpu as pltpu

N_DEV = 4
EPS = 1e-5


def kernel(x, Wp):
    b, h_loc, w, c = x.shape
    c_out = Wp.shape[1]
    n_global = N_DEV * h_loc * w

    def body(x_ref, wp_ref, out_ref, stats_ref, send_sems, recv_sems):
        my_pos = lax.axis_index("i")

        barrier_sem = pltpu.get_barrier_semaphore()
        for k in range(1, N_DEV):
            peer = (my_pos + k) % N_DEV
            pl.semaphore_signal(
                barrier_sem, inc=1,
                device_id=(peer,), device_id_type=pl.DeviceIdType.MESH,
            )

        xf = x_ref[...].astype(jnp.float32)
        x2d = xf.reshape(b, h_loc * w, c)
        s1 = jnp.sum(x2d, axis=1)
        s2 = jnp.sum(x2d * x2d, axis=1)
        stats_ref[0, :, :] = jnp.concatenate([s1, s2], axis=0)

        pl.semaphore_wait(barrier_sem, N_DEV - 1)

        rdmas = []
        for k in range(1, N_DEV):
            peer = (my_pos + k) % N_DEV
            rdma = pltpu.make_async_remote_copy(
                src_ref=stats_ref.at[0],
                dst_ref=stats_ref.at[k],
                send_sem=send_sems.at[k - 1],
                recv_sem=recv_sems.at[k - 1],
                device_id=(peer,),
                device_id_type=pl.DeviceIdType.MESH,
            )
            rdma.start()
            rdmas.append(rdma)
        x_bf = xf.astype(jnp.bfloat16)
        wp = wp_ref[...].astype(jnp.bfloat16)

        for rdma in rdmas:
            rdma.wait_send()
        for rdma in rdmas:
            rdma.wait_recv()

        tot = (
            stats_ref[0, :, :] + stats_ref[1, :, :]
            + stats_ref[2, :, :] + stats_ref[3, :, :]
        )
        mean = tot[:b, :] / n_global
        var = tot[b:, :] / n_global - mean * mean
        inv = lax.rsqrt(var + EPS)
        mean_bf = mean.astype(jnp.bfloat16).reshape(b, 1, 1, c)
        inv_bf = inv.astype(jnp.bfloat16).reshape(b, 1, 1, c)
        h = (x_bf - mean_bf) * inv_bf
        a = (h * jax.nn.sigmoid(h)).reshape(b * h_loc * w, c)
        o = jnp.dot(a, wp, preferred_element_type=jnp.float32)
        out_ref[...] = o.reshape(b, h_loc, w, c_out)

    return pl.pallas_call(
        body,
        out_shape=jax.ShapeDtypeStruct((b, h_loc, w, c_out), jnp.float32),
        in_specs=[
            pl.BlockSpec(memory_space=pltpu.VMEM),
            pl.BlockSpec(memory_space=pltpu.VMEM),
        ],
        out_specs=pl.BlockSpec(memory_space=pltpu.VMEM),
        scratch_shapes=[
            pltpu.VMEM((N_DEV, 2 * b, c), jnp.float32),
            pltpu.SemaphoreType.DMA((N_DEV - 1,)),
            pltpu.SemaphoreType.DMA((N_DEV - 1,)),
        ],
        compiler_params=pltpu.CompilerParams(collective_id=0),
    )(x, Wp)


# device time: 11029 ns/iter; 1.1031x vs baseline; 1.0613x over previous
import jax
import jax.numpy as jnp
from jax import lax
from jax.experimental import pallas as pl
from jax.experimental.pallas import tpu as pltpu

N_DEV = 4
EPS = 1e-5


def kernel(x, Wp):
    b, h_loc, w, c = x.shape
    c_out = Wp.shape[1]
    n_global = N_DEV * h_loc * w

    def body(x_ref, wp_ref, out_ref, stats_ref, send_sems, recv_sems):
        my_pos = lax.axis_index("i")

        barrier_sem = pltpu.get_barrier_semaphore()
        for k in range(1, N_DEV):
            peer = (my_pos + k) % N_DEV
            pl.semaphore_signal(
                barrier_sem, inc=1,
                device_id=(peer,), device_id_type=pl.DeviceIdType.MESH,
            )

        n_loc = h_loc * w
        x_bf = x_ref[...].astype(jnp.bfloat16)
        x_flat = x_bf.reshape(b * n_loc, c)
        rows = lax.broadcasted_iota(jnp.int32, (b, b * n_loc), 0)
        cols = lax.broadcasted_iota(jnp.int32, (b, b * n_loc), 1)
        sel = (cols // n_loc == rows).astype(jnp.bfloat16)
        s1 = jnp.dot(sel, x_flat, preferred_element_type=jnp.float32)
        s2 = jnp.dot(sel, x_flat * x_flat, preferred_element_type=jnp.float32)
        stats_ref[0, :, :] = jnp.concatenate([s1, s2], axis=0)

        pl.semaphore_wait(barrier_sem, N_DEV - 1)

        rdmas = []
        for k in range(1, N_DEV):
            peer = (my_pos + k) % N_DEV
            rdma = pltpu.make_async_remote_copy(
                src_ref=stats_ref.at[0],
                dst_ref=stats_ref.at[k],
                send_sem=send_sems.at[k - 1],
                recv_sem=recv_sems.at[k - 1],
                device_id=(peer,),
                device_id_type=pl.DeviceIdType.MESH,
            )
            rdma.start()
            rdmas.append(rdma)
        wp = wp_ref[...].astype(jnp.bfloat16)

        for rdma in rdmas:
            rdma.wait_send()
        for rdma in rdmas:
            rdma.wait_recv()

        tot = (
            stats_ref[0, :, :] + stats_ref[1, :, :]
            + stats_ref[2, :, :] + stats_ref[3, :, :]
        )
        mean = tot[:b, :] / n_global
        var = tot[b:, :] / n_global - mean * mean
        inv = lax.rsqrt(var + EPS)
        mean_bf = mean.astype(jnp.bfloat16).reshape(b, 1, 1, c)
        inv_bf = inv.astype(jnp.bfloat16).reshape(b, 1, 1, c)
        h = (x_bf - mean_bf) * inv_bf
        a = (h * jax.nn.sigmoid(h)).reshape(b * h_loc * w, c)
        o = jnp.dot(a, wp, preferred_element_type=jnp.float32)
        out_ref[...] = o.reshape(b, h_loc, w, c_out).astype(jnp.bfloat16)

    return pl.pallas_call(
        body,
        out_shape=jax.ShapeDtypeStruct((b, h_loc, w, c_out), jnp.bfloat16),
        in_specs=[
            pl.BlockSpec(memory_space=pltpu.VMEM),
            pl.BlockSpec(memory_space=pltpu.VMEM),
        ],
        out_specs=pl.BlockSpec(memory_space=pltpu.VMEM),
        scratch_shapes=[
            pltpu.VMEM((N_DEV, 2 * b, c), jnp.float32),
            pltpu.SemaphoreType.DMA((N_DEV - 1,)),
            pltpu.SemaphoreType.DMA((N_DEV - 1,)),
        ],
        compiler_params=pltpu.CompilerParams(collective_id=0),
    )(x, Wp)
